# baseline (speedup 1.0000x reference)
import functools

import jax
import jax.numpy as jnp
from jax import lax
from jax.experimental import pallas as pl
from jax.experimental.pallas import tpu as pltpu
from jax.experimental.pallas import tpu_sc as plsc

_THRESH = 0.5
_NEGPOS = 3.0
_VAR = 0.1
_CHUNK = 4096
_TILE = 4096
_LANE = 128


def _match_kernel(tc_ref, tbl_ref, pt_ref, conf_ref, g_ref, rt_ref,
                  btm_scr, bti_scr):
    O = tc_ref.shape[1]
    P = pt_ref.shape[1]
    nch = P // _CHUNK
    tc = tc_ref[0]
    ax1, ay1 = tc[:, 0:1], tc[:, 1:2]
    ax2, ay2 = tc[:, 2:3], tc[:, 3:4]
    area_a = (ax2 - ax1) * (ay2 - ay1)
    row_i = lax.broadcasted_iota(jnp.int32, (O, _CHUNK), 0)
    lane_i = lax.broadcasted_iota(jnp.int32, (O, _CHUNK), 1)

    bp_max = jnp.full((O, 1), -1.0, jnp.float32)
    bp_idx = jnp.zeros((O, 1), jnp.int32)
    for c in range(nch):
        sl = slice(c * _CHUNK, (c + 1) * _CHUNK)
        pcx, pcy = pt_ref[0:1, sl], pt_ref[1:2, sl]
        pw, ph = pt_ref[2:3, sl], pt_ref[3:4, sl]
        bx1, by1 = pcx - pw / 2, pcy - ph / 2
        bx2, by2 = pcx + pw / 2, pcy + ph / 2
        wid = jnp.maximum(jnp.minimum(ax2, bx2) - jnp.maximum(ax1, bx1), 0.0)
        hei = jnp.maximum(jnp.minimum(ay2, by2) - jnp.maximum(ay1, by1), 0.0)
        inter = wid * hei
        area_b = (bx2 - bx1) * (by2 - by1)
        ovl = inter / (area_a + area_b - inter)
        cmax = jnp.max(ovl, axis=1, keepdims=True)
        cidx = jnp.min(jnp.where(ovl == cmax, lane_i, P), axis=1,
                       keepdims=True) + c * _CHUNK
        upd = cmax > bp_max
        bp_idx = jnp.where(upd, cidx, bp_idx)
        bp_max = jnp.where(upd, cmax, bp_max)
        btm = jnp.max(ovl, axis=0, keepdims=True)
        bti = jnp.min(jnp.where(ovl == btm, row_i, O), axis=0, keepdims=True)
        btm_scr[0:1, sl] = btm
        bti_scr[0:1, sl] = bti

    tbl = tbl_ref[0]
    for c in range(nch):
        sl = slice(c * _CHUNK, (c + 1) * _CHUNK)
        btm = btm_scr[0:1, sl]
        bti = bti_scr[0:1, sl]
        lane_g = lane_i + c * _CHUNK
        F = lane_g == bp_idx
        ch_o = jnp.max(jnp.where(F, row_i, -1), axis=0, keepdims=True)
        forced = ch_o >= 0
        fidx = jnp.where(forced, ch_o, bti)
        fovl = jnp.where(forced, 2.0, btm)
        eq2f = (row_i == fidx).astype(jnp.float32)
        m = jax.lax.dot(tbl, eq2f,
                        precision=jax.lax.Precision.HIGHEST,
                        preferred_element_type=jnp.float32)
        conf = jnp.where(fovl < _THRESH, 0.0, m[0:1])
        pcx, pcy = pt_ref[0:1, sl], pt_ref[1:2, sl]
        pw, ph = pt_ref[2:3, sl], pt_ref[3:4, sl]
        conf_ref[0, 0:1, sl] = conf.astype(jnp.int32)
        g_ref[0, 0:1, sl] = (m[1:2] - pcx) / (_VAR * pw)
        g_ref[0, 1:2, sl] = (m[2:3] - pcy) / (_VAR * ph)
        rt_ref[0, 0:1, sl] = m[3:4]


def _stream_kernel(conf_ref, lx_ref, ly_ref, rd_ref, ct_ref, gx_ref, gy_ref,
                   rt_ref, ce_ref, acc_ref):
    b = pl.program_id(0)
    t = pl.program_id(1)
    G, L, C = conf_ref.shape[1], conf_ref.shape[2], conf_ref.shape[3]

    @pl.when(jnp.logical_and(b == 0, t == 0))
    def _init():
        acc_ref[...] = jnp.zeros_like(acc_ref)

    x = conf_ref[0]
    m = jnp.max(x, axis=2)
    s = jnp.sum(jnp.exp(x - m[:, :, None]), axis=2)
    lse = m + jnp.log(s)
    ct = ct_ref[0]
    cls_i = lax.broadcasted_iota(jnp.int32, (G, L, C), 2)
    gathered = jnp.sum(jnp.where(cls_i == ct[:, :, None], x, 0.0), axis=2)
    ce = lse - gathered
    pos = ct > 0
    posf = pos.astype(jnp.float32)
    ce_ref[0] = jnp.where(pos, 0.0, jnp.maximum(ce, 0.0))

    dx = lx_ref[0] - gx_ref[0]
    dy = ly_ref[0] - gy_ref[0]
    adx, ady = jnp.abs(dx), jnp.abs(dy)
    sl1 = (jnp.where(adx < 1.0, 0.5 * dx * dx, adx - 0.5)
           + jnp.where(ady < 1.0, 0.5 * dy * dy, ady - 0.5))
    part_l = jnp.sum(sl1 * posf)
    part_r = jnp.sum(jnp.abs(rd_ref[0] - rt_ref[0]) * posf)
    part_ce = jnp.sum(ce * posf)
    npos = jnp.sum(posf)

    li = lax.broadcasted_iota(jnp.int32, (1, 128), 1)
    v = (jnp.where(li == 0, part_l, 0.0) + jnp.where(li == 1, part_r, 0.0)
         + jnp.where(li == 2, part_ce, 0.0) + jnp.where(li == 3, npos, 0.0))
    ri = lax.broadcasted_iota(jnp.int32, (acc_ref.shape[0], 128), 0)
    acc_ref[...] += jnp.where(ri == b, 1.0, 0.0) * v


_L = 16
_NBIN = 2048


def _sc_select_kernel(ce_hbm, kacc_hbm, out_hbm, vals, hist, mem, tmp16):
    c = lax.axis_index("c")
    s = lax.axis_index("s")
    B = ce_hbm.shape[0]
    P = ce_hbm.shape[1]
    nchunk = P // _L
    lane = lax.broadcasted_iota(jnp.int32, (_L,), 0)

    @pl.when(jnp.logical_and(c == 0, s < B))
    def _row_work():
        pltpu.sync_copy(ce_hbm.at[s], vals)
        pltpu.sync_copy(kacc_hbm.at[3], tmp16)
        npos = tmp16[...]
        ks = jnp.max(jnp.where(lane == s,
                               jnp.minimum(_NEGPOS * npos, float(P - 1)),
                               -1.0))

        def zero_hist(j, _):
            hist[0, pl.ds(j * _L, _L)] = jnp.zeros((_L,), jnp.int32)
            hist[1, pl.ds(j * _L, _L)] = jnp.zeros((_L,), jnp.int32)
            hist[2, pl.ds(j * _L, _L)] = jnp.zeros((_L,), jnp.int32)
            hist[3, pl.ds(j * _L, _L)] = jnp.zeros((_L,), jnp.int32)
            hist[4, pl.ds(j * _L, _L)] = jnp.zeros((_L,), jnp.int32)
            hist[5, pl.ds(j * _L, _L)] = jnp.zeros((_L,), jnp.int32)
            hist[6, pl.ds(j * _L, _L)] = jnp.zeros((_L,), jnp.int32)
            hist[7, pl.ds(j * _L, _L)] = jnp.zeros((_L,), jnp.int32)
            hist[8, pl.ds(j * _L, _L)] = jnp.zeros((_L,), jnp.int32)
            hist[9, pl.ds(j * _L, _L)] = jnp.zeros((_L,), jnp.int32)
            hist[10, pl.ds(j * _L, _L)] = jnp.zeros((_L,), jnp.int32)
            hist[11, pl.ds(j * _L, _L)] = jnp.zeros((_L,), jnp.int32)
            hist[12, pl.ds(j * _L, _L)] = jnp.zeros((_L,), jnp.int32)
            hist[13, pl.ds(j * _L, _L)] = jnp.zeros((_L,), jnp.int32)
            hist[14, pl.ds(j * _L, _L)] = jnp.zeros((_L,), jnp.int32)
            hist[15, pl.ds(j * _L, _L)] = jnp.zeros((_L,), jnp.int32)
            return 0
        lax.fori_loop(0, _NBIN // _L, zero_hist, 0, unroll=2)

        ones = jnp.ones((_L,), jnp.int32)

        def hist_pass(j, _):
            v = vals[pl.ds(j * _L, _L)]
            b = lax.shift_right_logical(plsc.bitcast(v, jnp.int32), 20)
            plsc.addupdate_scatter(hist, [lane, b], ones)
            return 0
        lax.fori_loop(0, nchunk, hist_pass, 0, unroll=4)

        def scan_pass(jj, carry):
            cum, found, bstar, cntgt = carry
            j = (_NBIN // _L - 1) - jj
            sl = pl.ds(j * _L, _L)
            tot = (hist[0, sl] + hist[1, sl] + hist[2, sl] + hist[3, sl]
                   + hist[4, sl] + hist[5, sl] + hist[6, sl] + hist[7, sl]
                   + hist[8, sl] + hist[9, sl] + hist[10, sl] + hist[11, sl]
                   + hist[12, sl] + hist[13, sl] + hist[14, sl]
                   + hist[15, sl]).astype(jnp.float32)
            s_chunk = jnp.sum(tot)
            rev = lax.rev(tot, (0,))
            cs = plsc.cumsum(rev)
            mask = (cum + cs) >= ks
            m = plsc.all_reduce_ffs(mask)
            csm = jnp.sum(jnp.where(lane == m, cs, 0.0))
            revm = jnp.sum(jnp.where(lane == m, rev, 0.0))
            hit = jnp.logical_and(jnp.logical_not(found), cum + s_chunk >= ks)
            bstar = jnp.where(hit, j * _L + (_L - 1) - jnp.max(m), bstar)
            cntgt = jnp.where(hit, cum + csm - revm, cntgt)
            found = jnp.logical_or(found, hit)
            cum = jnp.where(found, cum, cum + s_chunk)
            return cum, found, bstar, cntgt
        _, _, bstar, cnt_gt_bin = lax.fori_loop(
            0, _NBIN // _L, scan_pass, (0.0, False, 0, 0.0))

        def sum_pass(j, carry):
            off, sgt = carry
            v = vals[pl.ds(j * _L, _L)]
            b = lax.shift_right_logical(plsc.bitcast(v, jnp.int32), 20)
            mgt = b > bstar
            sgt = sgt + jnp.sum(jnp.where(mgt, v, 0.0))
            meq = b == bstar
            pos = off + plsc.cumsum(meq.astype(jnp.int32)) - 1
            plsc.store_scatter(mem, [pos], v, mask=meq)
            off = off + jnp.sum(meq.astype(jnp.int32))
            return off, sgt
        moff, sum_gt_bin = lax.fori_loop(0, nchunk, sum_pass, (0, 0.0),
                                         unroll=4)

        mem[pl.ds(moff, _L)] = plsc.bitcast(
            jnp.full((_L,), -1, jnp.int32), jnp.float32)
        nmc = lax.shift_right_logical(moff + _L - 1, 4)

        kprime = ks - cnt_gt_bin

        def count_ge(mid):
            def cbody(j, cc):
                mb = plsc.bitcast(mem[pl.ds(j * _L, _L)], jnp.int32)
                return cc + jnp.sum(jnp.where(mb >= mid, 1.0, 0.0))
            return lax.fori_loop(0, nmc, cbody, 0.0)

        def bs_body(_, carry):
            lo, hi = carry
            mid = lo + lax.shift_right_logical(hi - lo + 1, 1)
            pred = count_ge(mid) >= kprime
            lo = jnp.where(pred, mid, lo)
            hi = jnp.where(pred, hi, mid - 1)
            return lo, hi
        lo0 = lax.shift_left(bstar, 20)
        hi0 = lo0 + (1 << 20) - 1
        thr, _ = lax.fori_loop(0, 20, bs_body, (lo0, hi0))
        thr_f = jnp.max(plsc.bitcast(
            jnp.full((_L,), 1, jnp.int32) * thr, jnp.float32))

        def final_body(j, carry):
            cgt, sgt = carry
            mv = mem[pl.ds(j * _L, _L)]
            mb = plsc.bitcast(mv, jnp.int32)
            m2 = mb > thr
            cgt = cgt + jnp.sum(jnp.where(m2, 1.0, 0.0))
            sgt = sgt + jnp.sum(jnp.where(m2, mv, 0.0))
            return cgt, sgt
        cnt_gt_thr, sum_gt_thr = lax.fori_loop(0, nmc, final_body, (0.0, 0.0))

        topk = (sum_gt_bin + sum_gt_thr
                + (ks - cnt_gt_bin - cnt_gt_thr) * thr_f)
        tmp16[...] = jnp.zeros((_L,), jnp.float32) + topk
        pltpu.sync_copy(tmp16, out_hbm.at[s])


def _combine_kernel(acc_ref, topk_ref, out_ref):
    accv = acc_ref[...]
    topks = topk_ref[...][:, 0:1]
    n_total = jnp.sum(accv[:, 3:4])
    loss_l = jnp.sum(accv[:, 0:1]) / n_total
    loss_r = jnp.sum(accv[:, 1:2]) / n_total
    loss_c = jnp.sum(accv[:, 2:3] + topks) / n_total
    ri = lax.broadcasted_iota(jnp.int32, out_ref.shape, 0)
    ci = lax.broadcasted_iota(jnp.int32, out_ref.shape, 1)
    r0 = ri == 0
    out_ref[...] = (jnp.where(r0 & (ci == 0), loss_l, 0.0)
                    + jnp.where(r0 & (ci == 1), loss_c, 0.0)
                    + jnp.where(r0 & (ci == 2), loss_r, 0.0))


def kernel(loc_data, conf_data, regr_data, priors, t_coords, t_labels, t_regr):
    B, P, C = conf_data.shape
    O = t_coords.shape[1]
    priors_t = priors.T
    tcx = (t_coords[:, :, 0] + t_coords[:, :, 2]) * 0.5
    tcy = (t_coords[:, :, 1] + t_coords[:, :, 3]) * 0.5
    tbl = jnp.stack([t_labels.astype(jnp.float32) + 1.0, tcx, tcy,
                     t_regr[:, :, 0]], axis=1)

    conf_t, g_row, rt_row = pl.pallas_call(
        _match_kernel,
        grid=(B,),
        in_specs=[
            pl.BlockSpec((1, O, 4), lambda b: (b, 0, 0)),
            pl.BlockSpec((1, 4, O), lambda b: (b, 0, 0)),
            pl.BlockSpec((4, P), lambda b: (0, 0)),
        ],
        out_specs=[
            pl.BlockSpec((1, 1, P), lambda b: (b, 0, 0)),
            pl.BlockSpec((1, 2, P), lambda b: (b, 0, 0)),
            pl.BlockSpec((1, 1, P), lambda b: (b, 0, 0)),
        ],
        out_shape=[
            jax.ShapeDtypeStruct((B, 1, P), jnp.int32),
            jax.ShapeDtypeStruct((B, 2, P), jnp.float32),
            jax.ShapeDtypeStruct((B, 1, P), jnp.float32),
        ],
        scratch_shapes=[
            pltpu.VMEM((8, P), jnp.float32),
            pltpu.VMEM((8, P), jnp.int32),
        ],
    )(t_coords, tbl, priors_t)

    PG = P // _LANE
    TG = _TILE // _LANE
    conf4 = conf_data.reshape(B, PG, _LANE, C)
    ct_g = conf_t.reshape(B, PG, _LANE)
    gx_g = g_row[:, 0, :].reshape(B, PG, _LANE)
    gy_g = g_row[:, 1, :].reshape(B, PG, _LANE)
    rt_g = rt_row.reshape(B, PG, _LANE)
    lx_g = loc_data[:, :, 0].reshape(B, PG, _LANE)
    ly_g = loc_data[:, :, 1].reshape(B, PG, _LANE)
    rd_g = regr_data.reshape(B, PG, _LANE)

    nt = P // _TILE
    spec3 = pl.BlockSpec((1, TG, _LANE), lambda b, t: (b, t, 0))
    ce_neg, acc = pl.pallas_call(
        _stream_kernel,
        grid=(B, nt),
        in_specs=[
            pl.BlockSpec((1, TG, _LANE, C), lambda b, t: (b, t, 0, 0)),
            spec3, spec3, spec3, spec3, spec3, spec3, spec3,
        ],
        out_specs=[
            pl.BlockSpec((1, TG, _LANE), lambda b, t: (b, t, 0)),
            pl.BlockSpec((B, 128), lambda b, t: (0, 0)),
        ],
        out_shape=[
            jax.ShapeDtypeStruct((B, PG, _LANE), jnp.float32),
            jax.ShapeDtypeStruct((B, 128), jnp.float32),
        ],
    )(conf4, lx_g, ly_g, rd_g, ct_g, gx_g, gy_g, rt_g)

    sc_sel = functools.partial(
        pl.kernel,
        mesh=plsc.VectorSubcoreMesh(core_axis_name="c", subcore_axis_name="s"),
        out_type=jax.ShapeDtypeStruct((B, _L), jnp.float32),
        scratch_types=[
            pltpu.VMEM((P,), jnp.float32),
            pltpu.VMEM((_L, _NBIN), jnp.int32),
            pltpu.VMEM((P + _L,), jnp.float32),
            pltpu.VMEM((_L,), jnp.float32),
        ],
        compiler_params=pltpu.CompilerParams(needs_layout_passes=False),
    )(_sc_select_kernel)
    topk = sc_sel(ce_neg.reshape(B, P), acc[:, :4].T)

    out = pl.pallas_call(
        _combine_kernel,
        in_specs=[
            pl.BlockSpec((B, 128), lambda: (0, 0)),
            pl.BlockSpec((B, _L), lambda: (0, 0)),
        ],
        out_specs=pl.BlockSpec((8, 128), lambda: (0, 0)),
        out_shape=jax.ShapeDtypeStruct((8, 128), jnp.float32),
    )(acc, topk)

    return (out[0, 0], out[0, 1], out[0, 2])


# speedup vs baseline: 1.0024x; 1.0024x over previous
import functools

import jax
import jax.numpy as jnp
from jax import lax
from jax.experimental import pallas as pl
from jax.experimental.pallas import tpu as pltpu
from jax.experimental.pallas import tpu_sc as plsc

_THRESH = 0.5
_NEGPOS = 3.0
_VAR = 0.1
_CHUNK = 8192
_TILE = 8192
_LANE = 128


def _match_kernel(tc_ref, tbl_ref, pt_ref, conf_ref, g_ref, rt_ref,
                  btm_scr, bti_scr):
    O = tc_ref.shape[1]
    P = pt_ref.shape[1]
    nch = P // _CHUNK
    tc = tc_ref[0]
    ax1, ay1 = tc[:, 0:1], tc[:, 1:2]
    ax2, ay2 = tc[:, 2:3], tc[:, 3:4]
    area_a = (ax2 - ax1) * (ay2 - ay1)
    row_i = lax.broadcasted_iota(jnp.int32, (O, _CHUNK), 0)
    lane_i = lax.broadcasted_iota(jnp.int32, (O, _CHUNK), 1)

    bp_max = jnp.full((O, 1), -1.0, jnp.float32)
    bp_idx = jnp.zeros((O, 1), jnp.int32)
    for c in range(nch):
        sl = slice(c * _CHUNK, (c + 1) * _CHUNK)
        pcx, pcy = pt_ref[0:1, sl], pt_ref[1:2, sl]
        pw, ph = pt_ref[2:3, sl], pt_ref[3:4, sl]
        bx1, by1 = pcx - pw / 2, pcy - ph / 2
        bx2, by2 = pcx + pw / 2, pcy + ph / 2
        wid = jnp.maximum(jnp.minimum(ax2, bx2) - jnp.maximum(ax1, bx1), 0.0)
        hei = jnp.maximum(jnp.minimum(ay2, by2) - jnp.maximum(ay1, by1), 0.0)
        inter = wid * hei
        area_b = (bx2 - bx1) * (by2 - by1)
        ovl = inter / (area_a + area_b - inter)
        cmax = jnp.max(ovl, axis=1, keepdims=True)
        cidx = jnp.min(jnp.where(ovl == cmax, lane_i, P), axis=1,
                       keepdims=True) + c * _CHUNK
        upd = cmax > bp_max
        bp_idx = jnp.where(upd, cidx, bp_idx)
        bp_max = jnp.where(upd, cmax, bp_max)
        btm = jnp.max(ovl, axis=0, keepdims=True)
        bti = jnp.min(jnp.where(ovl == btm, row_i, O), axis=0, keepdims=True)
        btm_scr[0:1, sl] = btm
        bti_scr[0:1, sl] = bti

    tbl = tbl_ref[0]
    for c in range(nch):
        sl = slice(c * _CHUNK, (c + 1) * _CHUNK)
        btm = btm_scr[0:1, sl]
        bti = bti_scr[0:1, sl]
        lane_g = lane_i + c * _CHUNK
        F = lane_g == bp_idx
        ch_o = jnp.max(jnp.where(F, row_i, -1), axis=0, keepdims=True)
        forced = ch_o >= 0
        fidx = jnp.where(forced, ch_o, bti)
        fovl = jnp.where(forced, 2.0, btm)
        eq2f = (row_i == fidx).astype(jnp.float32)
        m = jax.lax.dot(tbl, eq2f,
                        precision=jax.lax.Precision.HIGHEST,
                        preferred_element_type=jnp.float32)
        conf = jnp.where(fovl < _THRESH, 0.0, m[0:1])
        pcx, pcy = pt_ref[0:1, sl], pt_ref[1:2, sl]
        pw, ph = pt_ref[2:3, sl], pt_ref[3:4, sl]
        conf_ref[0, 0:1, sl] = conf.astype(jnp.int32)
        g_ref[0, 0:1, sl] = (m[1:2] - pcx) / (_VAR * pw)
        g_ref[0, 1:2, sl] = (m[2:3] - pcy) / (_VAR * ph)
        rt_ref[0, 0:1, sl] = m[3:4]


def _stream_kernel(conf_ref, lx_ref, ly_ref, rd_ref, ct_ref, gx_ref, gy_ref,
                   rt_ref, ce_ref, acc_ref):
    b = pl.program_id(0)
    t = pl.program_id(1)
    G, L, C = conf_ref.shape[1], conf_ref.shape[2], conf_ref.shape[3]

    @pl.when(jnp.logical_and(b == 0, t == 0))
    def _init():
        acc_ref[...] = jnp.zeros_like(acc_ref)

    x = conf_ref[0]
    m = jnp.max(x, axis=2)
    s = jnp.sum(jnp.exp(x - m[:, :, None]), axis=2)
    lse = m + jnp.log(s)
    ct = ct_ref[0]
    cls_i = lax.broadcasted_iota(jnp.int32, (G, L, C), 2)
    gathered = jnp.sum(jnp.where(cls_i == ct[:, :, None], x, 0.0), axis=2)
    ce = lse - gathered
    pos = ct > 0
    posf = pos.astype(jnp.float32)
    ce_ref[0] = jnp.where(pos, 0.0, jnp.maximum(ce, 0.0))

    dx = lx_ref[0] - gx_ref[0]
    dy = ly_ref[0] - gy_ref[0]
    adx, ady = jnp.abs(dx), jnp.abs(dy)
    sl1 = (jnp.where(adx < 1.0, 0.5 * dx * dx, adx - 0.5)
           + jnp.where(ady < 1.0, 0.5 * dy * dy, ady - 0.5))
    part_l = jnp.sum(sl1 * posf)
    part_r = jnp.sum(jnp.abs(rd_ref[0] - rt_ref[0]) * posf)
    part_ce = jnp.sum(ce * posf)
    npos = jnp.sum(posf)

    li = lax.broadcasted_iota(jnp.int32, (1, 128), 1)
    v = (jnp.where(li == 0, part_l, 0.0) + jnp.where(li == 1, part_r, 0.0)
         + jnp.where(li == 2, part_ce, 0.0) + jnp.where(li == 3, npos, 0.0))
    ri = lax.broadcasted_iota(jnp.int32, (acc_ref.shape[0], 128), 0)
    acc_ref[...] += jnp.where(ri == b, 1.0, 0.0) * v


_L = 16
_NBIN = 2048


def _sc_select_kernel(ce_hbm, kacc_hbm, out_hbm, vals, hist, mem, tmp16):
    c = lax.axis_index("c")
    s = lax.axis_index("s")
    B = ce_hbm.shape[0]
    P = ce_hbm.shape[1]
    nchunk = P // _L
    lane = lax.broadcasted_iota(jnp.int32, (_L,), 0)

    @pl.when(jnp.logical_and(c == 0, s < B))
    def _row_work():
        pltpu.sync_copy(ce_hbm.at[s], vals)
        pltpu.sync_copy(kacc_hbm.at[3], tmp16)
        npos = tmp16[...]
        ks = jnp.max(jnp.where(lane == s,
                               jnp.minimum(_NEGPOS * npos, float(P - 1)),
                               -1.0))

        def zero_hist(j, _):
            hist[0, pl.ds(j * _L, _L)] = jnp.zeros((_L,), jnp.int32)
            hist[1, pl.ds(j * _L, _L)] = jnp.zeros((_L,), jnp.int32)
            hist[2, pl.ds(j * _L, _L)] = jnp.zeros((_L,), jnp.int32)
            hist[3, pl.ds(j * _L, _L)] = jnp.zeros((_L,), jnp.int32)
            hist[4, pl.ds(j * _L, _L)] = jnp.zeros((_L,), jnp.int32)
            hist[5, pl.ds(j * _L, _L)] = jnp.zeros((_L,), jnp.int32)
            hist[6, pl.ds(j * _L, _L)] = jnp.zeros((_L,), jnp.int32)
            hist[7, pl.ds(j * _L, _L)] = jnp.zeros((_L,), jnp.int32)
            hist[8, pl.ds(j * _L, _L)] = jnp.zeros((_L,), jnp.int32)
            hist[9, pl.ds(j * _L, _L)] = jnp.zeros((_L,), jnp.int32)
            hist[10, pl.ds(j * _L, _L)] = jnp.zeros((_L,), jnp.int32)
            hist[11, pl.ds(j * _L, _L)] = jnp.zeros((_L,), jnp.int32)
            hist[12, pl.ds(j * _L, _L)] = jnp.zeros((_L,), jnp.int32)
            hist[13, pl.ds(j * _L, _L)] = jnp.zeros((_L,), jnp.int32)
            hist[14, pl.ds(j * _L, _L)] = jnp.zeros((_L,), jnp.int32)
            hist[15, pl.ds(j * _L, _L)] = jnp.zeros((_L,), jnp.int32)
            return 0
        lax.fori_loop(0, _NBIN // _L, zero_hist, 0, unroll=2)

        ones = jnp.ones((_L,), jnp.int32)

        def hist_pass(j, _):
            v = vals[pl.ds(j * _L, _L)]
            b = lax.shift_right_logical(plsc.bitcast(v, jnp.int32), 20)
            plsc.addupdate_scatter(hist, [lane, b], ones)
            return 0
        lax.fori_loop(0, nchunk, hist_pass, 0, unroll=4)

        def scan_pass(jj, carry):
            cum, found, bstar, cntgt = carry
            j = (_NBIN // _L - 1) - jj
            sl = pl.ds(j * _L, _L)
            tot = (hist[0, sl] + hist[1, sl] + hist[2, sl] + hist[3, sl]
                   + hist[4, sl] + hist[5, sl] + hist[6, sl] + hist[7, sl]
                   + hist[8, sl] + hist[9, sl] + hist[10, sl] + hist[11, sl]
                   + hist[12, sl] + hist[13, sl] + hist[14, sl]
                   + hist[15, sl]).astype(jnp.float32)
            s_chunk = jnp.sum(tot)
            rev = lax.rev(tot, (0,))
            cs = plsc.cumsum(rev)
            mask = (cum + cs) >= ks
            m = plsc.all_reduce_ffs(mask)
            csm = jnp.sum(jnp.where(lane == m, cs, 0.0))
            revm = jnp.sum(jnp.where(lane == m, rev, 0.0))
            hit = jnp.logical_and(jnp.logical_not(found), cum + s_chunk >= ks)
            bstar = jnp.where(hit, j * _L + (_L - 1) - jnp.max(m), bstar)
            cntgt = jnp.where(hit, cum + csm - revm, cntgt)
            found = jnp.logical_or(found, hit)
            cum = jnp.where(found, cum, cum + s_chunk)
            return cum, found, bstar, cntgt
        _, _, bstar, cnt_gt_bin = lax.fori_loop(
            0, _NBIN // _L, scan_pass, (0.0, False, 0, 0.0))

        def sum_pass(j, carry):
            off, sgt = carry
            v = vals[pl.ds(j * _L, _L)]
            b = lax.shift_right_logical(plsc.bitcast(v, jnp.int32), 20)
            mgt = b > bstar
            sgt = sgt + jnp.sum(jnp.where(mgt, v, 0.0))
            meq = b == bstar
            pos = off + plsc.cumsum(meq.astype(jnp.int32)) - 1
            plsc.store_scatter(mem, [pos], v, mask=meq)
            off = off + jnp.sum(meq.astype(jnp.int32))
            return off, sgt
        moff, sum_gt_bin = lax.fori_loop(0, nchunk, sum_pass, (0, 0.0),
                                         unroll=4)

        mem[pl.ds(moff, _L)] = plsc.bitcast(
            jnp.full((_L,), -1, jnp.int32), jnp.float32)
        nmc = lax.shift_right_logical(moff + _L - 1, 4)

        kprime = ks - cnt_gt_bin

        def count_ge(mid):
            def cbody(j, cc):
                mb = plsc.bitcast(mem[pl.ds(j * _L, _L)], jnp.int32)
                return cc + jnp.sum(jnp.where(mb >= mid, 1.0, 0.0))
            return lax.fori_loop(0, nmc, cbody, 0.0)

        def bs_body(_, carry):
            lo, hi = carry
            mid = lo + lax.shift_right_logical(hi - lo + 1, 1)
            pred = count_ge(mid) >= kprime
            lo = jnp.where(pred, mid, lo)
            hi = jnp.where(pred, hi, mid - 1)
            return lo, hi
        lo0 = lax.shift_left(bstar, 20)
        hi0 = lo0 + (1 << 20) - 1
        thr, _ = lax.fori_loop(0, 20, bs_body, (lo0, hi0))
        thr_f = jnp.max(plsc.bitcast(
            jnp.full((_L,), 1, jnp.int32) * thr, jnp.float32))

        def final_body(j, carry):
            cgt, sgt = carry
            mv = mem[pl.ds(j * _L, _L)]
            mb = plsc.bitcast(mv, jnp.int32)
            m2 = mb > thr
            cgt = cgt + jnp.sum(jnp.where(m2, 1.0, 0.0))
            sgt = sgt + jnp.sum(jnp.where(m2, mv, 0.0))
            return cgt, sgt
        cnt_gt_thr, sum_gt_thr = lax.fori_loop(0, nmc, final_body, (0.0, 0.0))

        topk = (sum_gt_bin + sum_gt_thr
                + (ks - cnt_gt_bin - cnt_gt_thr) * thr_f)
        tmp16[...] = jnp.zeros((_L,), jnp.float32) + topk
        pltpu.sync_copy(tmp16, out_hbm.at[s])


def _combine_kernel(acc_ref, topk_ref, out_ref):
    accv = acc_ref[...]
    topks = topk_ref[...][:, 0:1]
    n_total = jnp.sum(accv[:, 3:4])
    loss_l = jnp.sum(accv[:, 0:1]) / n_total
    loss_r = jnp.sum(accv[:, 1:2]) / n_total
    loss_c = jnp.sum(accv[:, 2:3] + topks) / n_total
    ri = lax.broadcasted_iota(jnp.int32, out_ref.shape, 0)
    ci = lax.broadcasted_iota(jnp.int32, out_ref.shape, 1)
    r0 = ri == 0
    out_ref[...] = (jnp.where(r0 & (ci == 0), loss_l, 0.0)
                    + jnp.where(r0 & (ci == 1), loss_c, 0.0)
                    + jnp.where(r0 & (ci == 2), loss_r, 0.0))


def kernel(loc_data, conf_data, regr_data, priors, t_coords, t_labels, t_regr):
    B, P, C = conf_data.shape
    O = t_coords.shape[1]
    priors_t = priors.T
    tcx = (t_coords[:, :, 0] + t_coords[:, :, 2]) * 0.5
    tcy = (t_coords[:, :, 1] + t_coords[:, :, 3]) * 0.5
    tbl = jnp.stack([t_labels.astype(jnp.float32) + 1.0, tcx, tcy,
                     t_regr[:, :, 0]], axis=1)

    conf_t, g_row, rt_row = pl.pallas_call(
        _match_kernel,
        grid=(B,),
        in_specs=[
            pl.BlockSpec((1, O, 4), lambda b: (b, 0, 0)),
            pl.BlockSpec((1, 4, O), lambda b: (b, 0, 0)),
            pl.BlockSpec((4, P), lambda b: (0, 0)),
        ],
        out_specs=[
            pl.BlockSpec((1, 1, P), lambda b: (b, 0, 0)),
            pl.BlockSpec((1, 2, P), lambda b: (b, 0, 0)),
            pl.BlockSpec((1, 1, P), lambda b: (b, 0, 0)),
        ],
        out_shape=[
            jax.ShapeDtypeStruct((B, 1, P), jnp.int32),
            jax.ShapeDtypeStruct((B, 2, P), jnp.float32),
            jax.ShapeDtypeStruct((B, 1, P), jnp.float32),
        ],
        scratch_shapes=[
            pltpu.VMEM((8, P), jnp.float32),
            pltpu.VMEM((8, P), jnp.int32),
        ],
    )(t_coords, tbl, priors_t)

    PG = P // _LANE
    TG = _TILE // _LANE
    conf4 = conf_data.reshape(B, PG, _LANE, C)
    ct_g = conf_t.reshape(B, PG, _LANE)
    gx_g = g_row[:, 0, :].reshape(B, PG, _LANE)
    gy_g = g_row[:, 1, :].reshape(B, PG, _LANE)
    rt_g = rt_row.reshape(B, PG, _LANE)
    lx_g = loc_data[:, :, 0].reshape(B, PG, _LANE)
    ly_g = loc_data[:, :, 1].reshape(B, PG, _LANE)
    rd_g = regr_data.reshape(B, PG, _LANE)

    nt = P // _TILE
    spec3 = pl.BlockSpec((1, TG, _LANE), lambda b, t: (b, t, 0))
    ce_neg, acc = pl.pallas_call(
        _stream_kernel,
        grid=(B, nt),
        in_specs=[
            pl.BlockSpec((1, TG, _LANE, C), lambda b, t: (b, t, 0, 0)),
            spec3, spec3, spec3, spec3, spec3, spec3, spec3,
        ],
        out_specs=[
            pl.BlockSpec((1, TG, _LANE), lambda b, t: (b, t, 0)),
            pl.BlockSpec((B, 128), lambda b, t: (0, 0)),
        ],
        out_shape=[
            jax.ShapeDtypeStruct((B, PG, _LANE), jnp.float32),
            jax.ShapeDtypeStruct((B, 128), jnp.float32),
        ],
    )(conf4, lx_g, ly_g, rd_g, ct_g, gx_g, gy_g, rt_g)

    sc_sel = functools.partial(
        pl.kernel,
        mesh=plsc.VectorSubcoreMesh(core_axis_name="c", subcore_axis_name="s"),
        out_type=jax.ShapeDtypeStruct((B, _L), jnp.float32),
        scratch_types=[
            pltpu.VMEM((P,), jnp.float32),
            pltpu.VMEM((_L, _NBIN), jnp.int32),
            pltpu.VMEM((P + _L,), jnp.float32),
            pltpu.VMEM((_L,), jnp.float32),
        ],
        compiler_params=pltpu.CompilerParams(needs_layout_passes=False),
    )(_sc_select_kernel)
    topk = sc_sel(ce_neg.reshape(B, P), acc[:, :4].T)

    out = pl.pallas_call(
        _combine_kernel,
        in_specs=[
            pl.BlockSpec((B, 128), lambda: (0, 0)),
            pl.BlockSpec((B, _L), lambda: (0, 0)),
        ],
        out_specs=pl.BlockSpec((8, 128), lambda: (0, 0)),
        out_shape=jax.ShapeDtypeStruct((8, 128), jnp.float32),
    )(acc, topk)

    return (out[0, 0], out[0, 1], out[0, 2])
